# baseline (speedup 1.0000x reference)
import functools

import jax
import jax.numpy as jnp
from jax import lax
from jax.experimental import pallas as pl
from jax.experimental.pallas import tpu as pltpu
from jax.experimental.pallas import tpu_sc as plsc

N = 10000
E = 320000
D = 128

NC, NS, L = 2, 16, 16
NW = NC * NS
EPT = E // NW
CH = 100
NCHUNK = EPT // CH
HCH = NCHUNK // 2
NPAD = 10240
RPT = NPAD // NS
BM = 1000
GRID = N // BM

_MESH = plsc.VectorSubcoreMesh(
    core_axis_name="c", subcore_axis_name="s", num_cores=NC, num_subcores=NS
)


def _fill1(ref, n, val):
    def body(i, _):
        ref[pl.ds(i * L, L)] = jnp.full((L,), val, jnp.float32)
        return 0
    lax.fori_loop(0, n // L, body, 0)


def _fill2(ref, rows, val):
    def body(i, _):
        for k in range(8):
            ref[i, pl.ds(k * L, L)] = jnp.full((L,), val, jnp.float32)
        return 0
    lax.fori_loop(0, rows, body, 0)


DCH = 125
DNCH = EPT // DCH


def _deg_body(dst_hbm, deg_out, dstv, ones, zbuf, deg_sh):
    c = lax.axis_index("c")
    s = lax.axis_index("s")
    wid = c * NS + s
    pltpu.sync_copy(dst_hbm.at[wid], dstv)
    _fill1(zbuf, RPT, 0.0)
    pltpu.sync_copy(zbuf, deg_sh.at[pl.ds(s * RPT, RPT)])
    _fill1(ones, 128, 1.0)
    plsc.subcore_barrier()

    def chunk(j, _):
        pltpu.sync_copy(ones.at[pl.ds(0, DCH)],
                        deg_sh.at[dstv.at[j]], add=True)
        return 0
    lax.fori_loop(0, DNCH, chunk, 0)
    plsc.subcore_barrier()
    pltpu.sync_copy(deg_sh.at[pl.ds(s * RPT, RPT)],
                    deg_out.at[c, pl.ds(s * RPT, RPT)])


_deg_call = pl.kernel(
    _deg_body,
    out_type=jax.ShapeDtypeStruct((NC, NPAD), jnp.float32),
    mesh=_MESH,
    scratch_types=[
        pltpu.VMEM((DNCH, DCH), jnp.int32),
        pltpu.VMEM((128,), jnp.float32),
        pltpu.VMEM((RPT,), jnp.float32),
        pltpu.VMEM_SHARED((NPAD,), jnp.float32),
    ],
)


def _agg_body(g_hbm, src_hbm, dst_hbm, dum_hbm, agg_out,
              srcv, dstv, rows_a, rows_b, agg_sh, sem_a, sem_b):
    c = lax.axis_index("c")
    s = lax.axis_index("s")
    wid = c * NS + s
    pltpu.sync_copy(src_hbm.at[wid], srcv)
    pltpu.sync_copy(dst_hbm.at[wid, 0], dstv)
    _fill2(rows_a, CH, 0.0)
    for k in range(RPT // CH):
        pltpu.sync_copy(rows_a, agg_sh.at[pl.ds(s * RPT + k * CH, CH)])
    pltpu.sync_copy(rows_a.at[pl.ds(0, RPT - CH * (RPT // CH))],
                    agg_sh.at[pl.ds(s * RPT + CH * (RPT // CH),
                                    RPT - CH * (RPT // CH))])
    plsc.subcore_barrier()

    def gather(j, rows, sem):
        pltpu.async_copy(g_hbm.at[srcv.at[j]], rows, sem)

    def wait(rows, sem):
        pltpu.make_async_copy(dum_hbm, rows, sem).wait()

    gather(0, rows_a, sem_a)

    def chunk_pair(h):
        def body(t, _):
            j0 = h * HCH + 2 * t
            gather(j0 + 1, rows_b, sem_b)
            wait(rows_a, sem_a)
            pltpu.sync_copy(rows_a, agg_sh.at[dstv.at[2 * t]], add=True)
            gather(j0 + 2, rows_a, sem_a)
            wait(rows_b, sem_b)
            pltpu.sync_copy(rows_b, agg_sh.at[dstv.at[2 * t + 1]], add=True)
            return 0
        return body

    lax.fori_loop(0, HCH // 2, chunk_pair(0), 0)
    pltpu.sync_copy(dst_hbm.at[wid, 1], dstv)
    lax.fori_loop(0, HCH // 2 - 1, chunk_pair(1), 0)
    gather(NCHUNK - 1, rows_b, sem_b)
    wait(rows_a, sem_a)
    pltpu.sync_copy(rows_a, agg_sh.at[dstv.at[HCH - 2]], add=True)
    wait(rows_b, sem_b)
    pltpu.sync_copy(rows_b, agg_sh.at[dstv.at[HCH - 1]], add=True)

    plsc.subcore_barrier()
    pltpu.sync_copy(agg_sh.at[pl.ds(s * RPT, RPT)],
                    agg_out.at[c, pl.ds(s * RPT, RPT)])


_agg_call = pl.kernel(
    _agg_body,
    out_type=jax.ShapeDtypeStruct((NC, NPAD, D), jnp.float32),
    mesh=_MESH,
    scratch_types=[
        pltpu.VMEM((NCHUNK, CH), jnp.int32),
        pltpu.VMEM((HCH, CH), jnp.int32),
        pltpu.VMEM((CH, D), jnp.float32),
        pltpu.VMEM((CH, D), jnp.float32),
        pltpu.VMEM_SHARED((NPAD, D), jnp.float32),
        pltpu.SemaphoreType.DMA,
        pltpu.SemaphoreType.DMA,
    ],
)


def _mm1_body(x_ref, deg_ref, w1_ref, g1_ref):
    g1_ref[...] = lax.rsqrt(deg_ref[...]) * jnp.dot(
        x_ref[...], w1_ref[...], preferred_element_type=jnp.float32)


def _mm1(x, degc, W1):
    return pl.pallas_call(
        _mm1_body,
        grid=(GRID,),
        in_specs=[
            pl.BlockSpec((BM, D), lambda i: (i, 0)),
            pl.BlockSpec((BM, 1), lambda i: (i, 0)),
            pl.BlockSpec((D, D), lambda i: (0, 0)),
        ],
        out_specs=pl.BlockSpec((BM, D), lambda i: (i, 0)),
        out_shape=jax.ShapeDtypeStruct((N, D), jnp.float32),
    )(x, degc, W1)


def _mid_body(agg_ref, g1_ref, deg_ref, b1_ref, w2_ref, g2_ref):
    a = agg_ref[0] + agg_ref[1]
    dinv = lax.rsqrt(deg_ref[...])
    c1 = jnp.maximum(dinv * (a + g1_ref[...]) + b1_ref[...], 0.0)
    g2_ref[...] = dinv * jnp.dot(
        c1, w2_ref[...], preferred_element_type=jnp.float32)


def _mid(agg1, g1, degc, b12, W2):
    return pl.pallas_call(
        _mid_body,
        grid=(GRID,),
        in_specs=[
            pl.BlockSpec((NC, BM, D), lambda i: (0, i, 0)),
            pl.BlockSpec((BM, D), lambda i: (i, 0)),
            pl.BlockSpec((BM, 1), lambda i: (i, 0)),
            pl.BlockSpec((1, D), lambda i: (0, 0)),
            pl.BlockSpec((D, D), lambda i: (0, 0)),
        ],
        out_specs=pl.BlockSpec((BM, D), lambda i: (i, 0)),
        out_shape=jax.ShapeDtypeStruct((N, D), jnp.float32),
    )(agg1, g1, degc, b12, W2)


def _out_body(agg_ref, g2_ref, deg_ref, b2_ref, x_ref, wfc_ref, bfc_ref,
              out_ref):
    a = agg_ref[0] + agg_ref[1]
    res = jnp.dot(
        x_ref[...], wfc_ref[...], preferred_element_type=jnp.float32
    ) + bfc_ref[...]
    out_ref[...] = jnp.maximum(
        lax.rsqrt(deg_ref[...]) * (a + g2_ref[...]) + b2_ref[...], 0.0) + res


def _outk(agg2, g2, degc, b22, x, Wfc, bfc2):
    return pl.pallas_call(
        _out_body,
        grid=(GRID,),
        in_specs=[
            pl.BlockSpec((NC, BM, D), lambda i: (0, i, 0)),
            pl.BlockSpec((BM, D), lambda i: (i, 0)),
            pl.BlockSpec((BM, 1), lambda i: (i, 0)),
            pl.BlockSpec((1, D), lambda i: (0, 0)),
            pl.BlockSpec((BM, D), lambda i: (i, 0)),
            pl.BlockSpec((D, D), lambda i: (0, 0)),
            pl.BlockSpec((1, D), lambda i: (0, 0)),
        ],
        out_specs=pl.BlockSpec((BM, D), lambda i: (i, 0)),
        out_shape=jax.ShapeDtypeStruct((N, D), jnp.float32),
    )(agg2, g2, degc, b22, x, Wfc, bfc2)


@jax.jit
def _run(x, edge_index, W1, b1, W2, b2, Wfc, bfc):
    src = edge_index[0].reshape(NW, NCHUNK, CH)
    dst = edge_index[1].reshape(NW, 2, HCH, CH)
    dstd = edge_index[1].reshape(NW, DNCH, DCH)

    deg_parts = _deg_call(dstd)
    degc = (deg_parts[0, :N] + deg_parts[1, :N] + 1.0).reshape(N, 1)

    g1 = _mm1(x, degc, W1)
    dum = jnp.zeros((CH, D), jnp.float32)
    agg1 = _agg_call(g1, src, dst, dum)
    g2 = _mid(agg1, g1, degc, b1.reshape(1, D), W2)
    agg2 = _agg_call(g2, src, dst, dum)
    return _outk(agg2, g2, degc, b2.reshape(1, D), x, Wfc,
                 bfc.reshape(1, D))


def kernel(x, edge_index, W1, b1, W2, b2, Wfc, bfc):
    return _run(x, edge_index, W1, b1, W2, b2, Wfc, bfc)


# speedup vs baseline: 1.0020x; 1.0020x over previous
import functools

import jax
import jax.numpy as jnp
from jax import lax
from jax.experimental import pallas as pl
from jax.experimental.pallas import tpu as pltpu
from jax.experimental.pallas import tpu_sc as plsc

N = 10000
E = 320000
D = 128

NC, NS, L = 2, 16, 16
NW = NC * NS
EPT = E // NW
CH = 100
NCHUNK = EPT // CH
HCH = NCHUNK // 2
NPAD = 10240
RPT = NPAD // NS
BM = 1000
GRID = N // BM

_MESH = plsc.VectorSubcoreMesh(
    core_axis_name="c", subcore_axis_name="s", num_cores=NC, num_subcores=NS
)


def _fill1(ref, n, val):
    def body(i, _):
        ref[pl.ds(i * L, L)] = jnp.full((L,), val, jnp.float32)
        return 0
    lax.fori_loop(0, n // L, body, 0)


def _fill2(ref, rows, val):
    def body(i, _):
        for k in range(8):
            ref[i, pl.ds(k * L, L)] = jnp.full((L,), val, jnp.float32)
        return 0
    lax.fori_loop(0, rows, body, 0)


DCH = 125
DNCH = EPT // DCH


def _deg_body(dst_hbm, deg_out, dstv, ones, zbuf, deg_sh):
    c = lax.axis_index("c")
    s = lax.axis_index("s")
    wid = c * NS + s
    pltpu.sync_copy(dst_hbm.at[wid], dstv)
    _fill1(zbuf, RPT, 0.0)
    pltpu.sync_copy(zbuf, deg_sh.at[pl.ds(s * RPT, RPT)])
    _fill1(ones, 128, 1.0)
    plsc.subcore_barrier()

    def chunk(j, _):
        pltpu.sync_copy(ones.at[pl.ds(0, DCH)],
                        deg_sh.at[dstv.at[j]], add=True)
        return 0
    lax.fori_loop(0, DNCH, chunk, 0)
    plsc.subcore_barrier()
    pltpu.sync_copy(deg_sh.at[pl.ds(s * RPT, RPT)],
                    deg_out.at[c, pl.ds(s * RPT, RPT)])


_deg_call = pl.kernel(
    _deg_body,
    out_type=jax.ShapeDtypeStruct((NC, NPAD), jnp.float32),
    mesh=_MESH,
    scratch_types=[
        pltpu.VMEM((DNCH, DCH), jnp.int32),
        pltpu.VMEM((128,), jnp.float32),
        pltpu.VMEM((RPT,), jnp.float32),
        pltpu.VMEM_SHARED((NPAD,), jnp.float32),
    ],
)


def _agg_body(g_hbm, src_hbm, dst_hbm, dum_hbm, agg_out,
              srcv, dstv, rows_a, rows_b, agg_sh, sem_a, sem_b):
    c = lax.axis_index("c")
    s = lax.axis_index("s")
    wid = c * NS + s
    pltpu.sync_copy(src_hbm.at[wid], srcv)
    pltpu.sync_copy(dst_hbm.at[wid, 0], dstv)
    _fill2(rows_a, CH, 0.0)
    for k in range(RPT // CH):
        pltpu.sync_copy(rows_a, agg_sh.at[pl.ds(s * RPT + k * CH, CH)])
    pltpu.sync_copy(rows_a.at[pl.ds(0, RPT - CH * (RPT // CH))],
                    agg_sh.at[pl.ds(s * RPT + CH * (RPT // CH),
                                    RPT - CH * (RPT // CH))])
    plsc.subcore_barrier()

    def gather(j, rows, sem):
        pltpu.async_copy(g_hbm.at[srcv.at[j]], rows, sem)

    def wait(rows, sem):
        pltpu.make_async_copy(dum_hbm, rows, sem).wait()

    gather(0, rows_a, sem_a)

    def chunk_pair(h):
        def body(t, _):
            j0 = h * HCH + 2 * t
            gather(j0 + 1, rows_b, sem_b)
            wait(rows_a, sem_a)
            pltpu.sync_copy(rows_a, agg_sh.at[dstv.at[2 * t]], add=True)
            gather(j0 + 2, rows_a, sem_a)
            wait(rows_b, sem_b)
            pltpu.sync_copy(rows_b, agg_sh.at[dstv.at[2 * t + 1]], add=True)
            return 0
        return body

    lax.fori_loop(0, HCH // 2, chunk_pair(0), 0)
    pltpu.sync_copy(dst_hbm.at[wid, 1], dstv)
    lax.fori_loop(0, HCH // 2 - 1, chunk_pair(1), 0)
    gather(NCHUNK - 1, rows_b, sem_b)
    wait(rows_a, sem_a)
    pltpu.sync_copy(rows_a, agg_sh.at[dstv.at[HCH - 2]], add=True)
    wait(rows_b, sem_b)
    pltpu.sync_copy(rows_b, agg_sh.at[dstv.at[HCH - 1]], add=True)

    plsc.subcore_barrier()
    pltpu.sync_copy(agg_sh.at[pl.ds(s * RPT, RPT)],
                    agg_out.at[c, pl.ds(s * RPT, RPT)])


_agg_call = pl.kernel(
    _agg_body,
    out_type=jax.ShapeDtypeStruct((NC, NPAD, D), jnp.float32),
    mesh=_MESH,
    scratch_types=[
        pltpu.VMEM((NCHUNK, CH), jnp.int32),
        pltpu.VMEM((HCH, CH), jnp.int32),
        pltpu.VMEM((CH, D), jnp.float32),
        pltpu.VMEM((CH, D), jnp.float32),
        pltpu.VMEM_SHARED((NPAD, D), jnp.float32),
        pltpu.SemaphoreType.DMA,
        pltpu.SemaphoreType.DMA,
    ],
)


def _mm1_body(x_ref, w1_ref, h1_ref):
    h1_ref[...] = jnp.dot(
        x_ref[...], w1_ref[...], preferred_element_type=jnp.float32)


def _mm1(x, W1):
    return pl.pallas_call(
        _mm1_body,
        grid=(GRID,),
        in_specs=[
            pl.BlockSpec((BM, D), lambda i: (i, 0)),
            pl.BlockSpec((D, D), lambda i: (0, 0)),
        ],
        out_specs=pl.BlockSpec((BM, D), lambda i: (i, 0)),
        out_shape=jax.ShapeDtypeStruct((N, D), jnp.float32),
    )(x, W1)


def _mid_body(agg_ref, g1_ref, deg_ref, b1_ref, w2_ref, g2_ref):
    a = agg_ref[0] + agg_ref[1]
    dinv = lax.rsqrt(deg_ref[...])
    c1 = jnp.maximum(dinv * (a + g1_ref[...]) + b1_ref[...], 0.0)
    g2_ref[...] = dinv * jnp.dot(
        c1, w2_ref[...], preferred_element_type=jnp.float32)


def _mid(agg1, g1, degc, b12, W2):
    return pl.pallas_call(
        _mid_body,
        grid=(GRID,),
        in_specs=[
            pl.BlockSpec((NC, BM, D), lambda i: (0, i, 0)),
            pl.BlockSpec((BM, D), lambda i: (i, 0)),
            pl.BlockSpec((BM, 1), lambda i: (i, 0)),
            pl.BlockSpec((1, D), lambda i: (0, 0)),
            pl.BlockSpec((D, D), lambda i: (0, 0)),
        ],
        out_specs=pl.BlockSpec((BM, D), lambda i: (i, 0)),
        out_shape=jax.ShapeDtypeStruct((N, D), jnp.float32),
    )(agg1, g1, degc, b12, W2)


def _out_body(agg_ref, g2_ref, deg_ref, b2_ref, x_ref, wfc_ref, bfc_ref,
              out_ref):
    a = agg_ref[0] + agg_ref[1]
    res = jnp.dot(
        x_ref[...], wfc_ref[...], preferred_element_type=jnp.float32
    ) + bfc_ref[...]
    out_ref[...] = jnp.maximum(
        lax.rsqrt(deg_ref[...]) * (a + g2_ref[...]) + b2_ref[...], 0.0) + res


def _outk(agg2, g2, degc, b22, x, Wfc, bfc2):
    return pl.pallas_call(
        _out_body,
        grid=(GRID,),
        in_specs=[
            pl.BlockSpec((NC, BM, D), lambda i: (0, i, 0)),
            pl.BlockSpec((BM, D), lambda i: (i, 0)),
            pl.BlockSpec((BM, 1), lambda i: (i, 0)),
            pl.BlockSpec((1, D), lambda i: (0, 0)),
            pl.BlockSpec((BM, D), lambda i: (i, 0)),
            pl.BlockSpec((D, D), lambda i: (0, 0)),
            pl.BlockSpec((1, D), lambda i: (0, 0)),
        ],
        out_specs=pl.BlockSpec((BM, D), lambda i: (i, 0)),
        out_shape=jax.ShapeDtypeStruct((N, D), jnp.float32),
    )(agg2, g2, degc, b22, x, Wfc, bfc2)


@jax.jit
def _run(x, edge_index, W1, b1, W2, b2, Wfc, bfc):
    src = edge_index[0].reshape(NW, NCHUNK, CH)
    dst = edge_index[1].reshape(NW, 2, HCH, CH)
    dstd = edge_index[1].reshape(NW, DNCH, DCH)

    deg_parts = _deg_call(dstd)
    h1 = _mm1(x, W1)
    degc = (deg_parts[0, :N] + deg_parts[1, :N] + 1.0).reshape(N, 1)

    g1 = lax.rsqrt(degc) * h1
    dum = jnp.zeros((CH, D), jnp.float32)
    agg1 = _agg_call(g1, src, dst, dum)
    g2 = _mid(agg1, g1, degc, b1.reshape(1, D), W2)
    agg2 = _agg_call(g2, src, dst, dum)
    return _outk(agg2, g2, degc, b2.reshape(1, D), x, Wfc,
                 bfc.reshape(1, D))


def kernel(x, edge_index, W1, b1, W2, b2, Wfc, bfc):
    return _run(x, edge_index, W1, b1, W2, b2, Wfc, bfc)


# speedup vs baseline: 1.0226x; 1.0206x over previous
import functools

import jax
import jax.numpy as jnp
from jax import lax
from jax.experimental import pallas as pl
from jax.experimental.pallas import tpu as pltpu
from jax.experimental.pallas import tpu_sc as plsc

N = 10000
E = 320000
D = 128

NC, NS, L = 2, 16, 16
NW = NC * NS
EPT = E // NW
CH = 100
NCHUNK = EPT // CH
HCH = NCHUNK // 2
NPAD = 10240
RPT = NPAD // NS
BM = 1000
GRID = N // BM

_MESH = plsc.VectorSubcoreMesh(
    core_axis_name="c", subcore_axis_name="s", num_cores=NC, num_subcores=NS
)


def _fill1(ref, n, val):
    def body(i, _):
        ref[pl.ds(i * L, L)] = jnp.full((L,), val, jnp.float32)
        return 0
    lax.fori_loop(0, n // L, body, 0)


def _fill2(ref, rows, val):
    def body(i, _):
        for k in range(8):
            ref[i, pl.ds(k * L, L)] = jnp.full((L,), val, jnp.float32)
        return 0
    lax.fori_loop(0, rows, body, 0)


DCH = 125
DNCH = EPT // DCH


def _deg_body(dst_hbm, deg_out, dstv, ones, zbuf, deg_sh):
    c = lax.axis_index("c")
    s = lax.axis_index("s")
    wid = c * NS + s
    pltpu.sync_copy(dst_hbm.at[wid], dstv)
    _fill1(zbuf, RPT, 0.0)
    pltpu.sync_copy(zbuf, deg_sh.at[pl.ds(s * RPT, RPT)])
    _fill1(ones, 128, 1.0)
    plsc.subcore_barrier()

    def chunk(j, _):
        pltpu.sync_copy(ones.at[pl.ds(0, DCH)],
                        deg_sh.at[dstv.at[j]], add=True)
        return 0
    lax.fori_loop(0, DNCH, chunk, 0)
    plsc.subcore_barrier()
    pltpu.sync_copy(deg_sh.at[pl.ds(s * RPT, RPT)],
                    deg_out.at[c, pl.ds(s * RPT, RPT)])


_deg_call = pl.kernel(
    _deg_body,
    out_type=jax.ShapeDtypeStruct((NC, NPAD), jnp.float32),
    mesh=_MESH,
    scratch_types=[
        pltpu.VMEM((DNCH, DCH), jnp.int32),
        pltpu.VMEM((128,), jnp.float32),
        pltpu.VMEM((RPT,), jnp.float32),
        pltpu.VMEM_SHARED((NPAD,), jnp.float32),
    ],
)


def _agg_body(g_hbm, src_hbm, dst_hbm, dum_hbm, agg_out,
              srcv, dstv, rows_a, rows_b, agg_sh, sem_a, sem_b):
    c = lax.axis_index("c")
    s = lax.axis_index("s")
    wid = c * NS + s
    zrem = RPT - CH * (RPT // CH)
    pltpu.async_copy(src_hbm.at[wid], srcv, sem_b)
    pltpu.async_copy(dst_hbm.at[wid, 0], dstv, sem_b)
    _fill2(rows_a, CH, 0.0)
    for k in range(RPT // CH):
        pltpu.async_copy(rows_a, agg_sh.at[pl.ds(s * RPT + k * CH, CH)],
                         sem_a)
    pltpu.async_copy(rows_a.at[pl.ds(0, zrem)],
                     agg_sh.at[pl.ds(s * RPT + CH * (RPT // CH), zrem)],
                     sem_a)
    pltpu.make_async_copy(src_hbm.at[wid], srcv, sem_b).wait()
    pltpu.make_async_copy(dst_hbm.at[wid, 0], dstv, sem_b).wait()
    for k in range(RPT // CH):
        pltpu.make_async_copy(rows_a, agg_sh.at[pl.ds(s * RPT + k * CH, CH)],
                              sem_a).wait()
    pltpu.make_async_copy(rows_a.at[pl.ds(0, zrem)],
                          agg_sh.at[pl.ds(s * RPT + CH * (RPT // CH), zrem)],
                          sem_a).wait()
    plsc.subcore_barrier()

    def gather(j, rows, sem):
        pltpu.async_copy(g_hbm.at[srcv.at[j]], rows, sem)

    def wait(rows, sem):
        pltpu.make_async_copy(dum_hbm, rows, sem).wait()

    gather(0, rows_a, sem_a)

    def chunk_pair(h):
        def body(t, _):
            j0 = h * HCH + 2 * t
            gather(j0 + 1, rows_b, sem_b)
            wait(rows_a, sem_a)
            pltpu.sync_copy(rows_a, agg_sh.at[dstv.at[2 * t]], add=True)
            gather(j0 + 2, rows_a, sem_a)
            wait(rows_b, sem_b)
            pltpu.sync_copy(rows_b, agg_sh.at[dstv.at[2 * t + 1]], add=True)
            return 0
        return body

    lax.fori_loop(0, HCH // 2, chunk_pair(0), 0)
    pltpu.sync_copy(dst_hbm.at[wid, 1], dstv)
    lax.fori_loop(0, HCH // 2 - 1, chunk_pair(1), 0)
    gather(NCHUNK - 1, rows_b, sem_b)
    wait(rows_a, sem_a)
    pltpu.sync_copy(rows_a, agg_sh.at[dstv.at[HCH - 2]], add=True)
    wait(rows_b, sem_b)
    pltpu.sync_copy(rows_b, agg_sh.at[dstv.at[HCH - 1]], add=True)

    plsc.subcore_barrier()
    pltpu.sync_copy(agg_sh.at[pl.ds(s * RPT, RPT)],
                    agg_out.at[c, pl.ds(s * RPT, RPT)])


_agg_call = pl.kernel(
    _agg_body,
    out_type=jax.ShapeDtypeStruct((NC, NPAD, D), jnp.float32),
    mesh=_MESH,
    scratch_types=[
        pltpu.VMEM((NCHUNK, CH), jnp.int32),
        pltpu.VMEM((HCH, CH), jnp.int32),
        pltpu.VMEM((CH, D), jnp.float32),
        pltpu.VMEM((CH, D), jnp.float32),
        pltpu.VMEM_SHARED((NPAD, D), jnp.float32),
        pltpu.SemaphoreType.DMA,
        pltpu.SemaphoreType.DMA,
    ],
)


def _mm1_body(x_ref, w1_ref, h1_ref):
    h1_ref[...] = jnp.dot(
        x_ref[...], w1_ref[...], preferred_element_type=jnp.float32)


def _mm1(x, W1):
    return pl.pallas_call(
        _mm1_body,
        grid=(GRID,),
        in_specs=[
            pl.BlockSpec((BM, D), lambda i: (i, 0)),
            pl.BlockSpec((D, D), lambda i: (0, 0)),
        ],
        out_specs=pl.BlockSpec((BM, D), lambda i: (i, 0)),
        out_shape=jax.ShapeDtypeStruct((N, D), jnp.float32),
    )(x, W1)


def _mid_body(agg_ref, g1_ref, deg_ref, b1_ref, w2_ref, g2_ref):
    a = agg_ref[0] + agg_ref[1]
    dinv = lax.rsqrt(deg_ref[...])
    c1 = jnp.maximum(dinv * (a + g1_ref[...]) + b1_ref[...], 0.0)
    g2_ref[...] = dinv * jnp.dot(
        c1, w2_ref[...], preferred_element_type=jnp.float32)


def _mid(agg1, g1, degc, b12, W2):
    return pl.pallas_call(
        _mid_body,
        grid=(GRID,),
        in_specs=[
            pl.BlockSpec((NC, BM, D), lambda i: (0, i, 0)),
            pl.BlockSpec((BM, D), lambda i: (i, 0)),
            pl.BlockSpec((BM, 1), lambda i: (i, 0)),
            pl.BlockSpec((1, D), lambda i: (0, 0)),
            pl.BlockSpec((D, D), lambda i: (0, 0)),
        ],
        out_specs=pl.BlockSpec((BM, D), lambda i: (i, 0)),
        out_shape=jax.ShapeDtypeStruct((N, D), jnp.float32),
    )(agg1, g1, degc, b12, W2)


def _out_body(agg_ref, g2_ref, deg_ref, b2_ref, x_ref, wfc_ref, bfc_ref,
              out_ref):
    a = agg_ref[0] + agg_ref[1]
    res = jnp.dot(
        x_ref[...], wfc_ref[...], preferred_element_type=jnp.float32
    ) + bfc_ref[...]
    out_ref[...] = jnp.maximum(
        lax.rsqrt(deg_ref[...]) * (a + g2_ref[...]) + b2_ref[...], 0.0) + res


def _outk(agg2, g2, degc, b22, x, Wfc, bfc2):
    return pl.pallas_call(
        _out_body,
        grid=(GRID,),
        in_specs=[
            pl.BlockSpec((NC, BM, D), lambda i: (0, i, 0)),
            pl.BlockSpec((BM, D), lambda i: (i, 0)),
            pl.BlockSpec((BM, 1), lambda i: (i, 0)),
            pl.BlockSpec((1, D), lambda i: (0, 0)),
            pl.BlockSpec((BM, D), lambda i: (i, 0)),
            pl.BlockSpec((D, D), lambda i: (0, 0)),
            pl.BlockSpec((1, D), lambda i: (0, 0)),
        ],
        out_specs=pl.BlockSpec((BM, D), lambda i: (i, 0)),
        out_shape=jax.ShapeDtypeStruct((N, D), jnp.float32),
    )(agg2, g2, degc, b22, x, Wfc, bfc2)


@jax.jit
def _run(x, edge_index, W1, b1, W2, b2, Wfc, bfc):
    src = edge_index[0].reshape(NW, NCHUNK, CH)
    dst = edge_index[1].reshape(NW, 2, HCH, CH)
    dstd = edge_index[1].reshape(NW, DNCH, DCH)

    deg_parts = _deg_call(dstd)
    h1 = _mm1(x, W1)
    degc = (deg_parts[0, :N] + deg_parts[1, :N] + 1.0).reshape(N, 1)

    g1 = lax.rsqrt(degc) * h1
    dum = jnp.zeros((CH, D), jnp.float32)
    agg1 = _agg_call(g1, src, dst, dum)
    g2 = _mid(agg1, g1, degc, b1.reshape(1, D), W2)
    agg2 = _agg_call(g2, src, dst, dum)
    return _outk(agg2, g2, degc, b2.reshape(1, D), x, Wfc,
                 bfc.reshape(1, D))


def kernel(x, edge_index, W1, b1, W2, b2, Wfc, bfc):
    return _run(x, edge_index, W1, b1, W2, b2, Wfc, bfc)
